# baseline (speedup 1.0000x reference)
import functools

import jax
import jax.numpy as jnp
from jax import lax
from jax.experimental import pallas as pl
from jax.experimental.pallas import tpu as pltpu
from jax.experimental.pallas import tpu_sc as plsc

PAD = 1
EPS = 1e-5

B, S, H = 4, 4096, 1024
NW = 32
TOK_W = (B * S) // NW
SEG_PER_ROW = S // TOK_W
CHUNK = 16
NCHUNK = TOK_W // CHUNK
NPAIR = NCHUNK // 2
UNROLL = 4


def _vfull(val, dtype=jnp.int32):
    return jnp.full((16,), val, dtype)


def _pad_mask(chunk):
    return jnp.minimum(jnp.abs(chunk - _vfull(PAD)), _vfull(1))


def _rsqrt16(x):
    i = plsc.bitcast(x, jnp.int32)
    i = _vfull(0x5F3759DF) - lax.shift_right_logical(i, _vfull(1))
    y = plsc.bitcast(i, jnp.float32)
    c15 = _vfull(1.5, jnp.float32)
    c05 = _vfull(0.5, jnp.float32)
    for _ in range(3):
        y = y * (c15 - c05 * x * y * y)
    return y


def _sc_body(ids_hbm, word_hbm, pos_hbm, ty_hbm, g_hbm, b_hbm, out_hbm,
             idsbuf, widx, pidx,
             wrows0, wrows1, prows0, prows1, obuf,
             tybuf, gbuf, bbuf,
             wsem0, wsem1, psem0, psem1, osem):
    wid = lax.axis_index("c") * 16 + lax.axis_index("s")
    row = wid // SEG_PER_ROW
    seg = wid % SEG_PER_ROW

    wrows = (wrows0, wrows1)
    prows = (prows0, prows1)
    wsem = (wsem0, wsem1)
    psem = (psem0, psem1)

    pltpu.sync_copy(ty_hbm, tybuf)
    pltpu.sync_copy(g_hbm, gbuf)
    pltpu.sync_copy(b_hbm, bbuf)

    pltpu.sync_copy(ids_hbm.at[pl.ds(row * (S // 16), S // 16)], idsbuf)

    def pref_body(j, acc):
        return acc + _pad_mask(idsbuf[j])

    accv = lax.fori_loop(0, seg * (TOK_W // 16), pref_body,
                         jnp.zeros((16,), jnp.int32))
    prefv = jnp.broadcast_to(jnp.sum(accv), (16,))

    segbase = seg * (TOK_W // 16)
    for c in range(NCHUNK):
        chunk = idsbuf[segbase + c]
        m = _pad_mask(chunk)
        incl = plsc.cumsum(m)
        widx[c] = chunk
        pidx[c] = (prefv + incl) * m + _vfull(PAD)
        prefv = prefv + jnp.broadcast_to(jnp.sum(m), (16,))

    base = wid * TOK_W
    tokv = lax.iota(jnp.int32, 16)
    onesv = _vfull(1)
    inv_h = _vfull(1.0 / H, jnp.float32)
    epsv = _vfull(EPS, jnp.float32)

    def issue(k, s):
        pltpu.async_copy(word_hbm.at[widx.at[k]], wrows[s], wsem[s])
        pltpu.async_copy(pos_hbm.at[pidx.at[k]], prows[s], psem[s])

    def wait_gather(k, s):
        pltpu.make_async_copy(word_hbm.at[widx.at[k]], wrows[s], wsem[s]).wait()
        pltpu.make_async_copy(pos_hbm.at[pidx.at[k]], prows[s], psem[s]).wait()

    def compute_chunk(w, p, o):
        zf = jnp.zeros((16,), jnp.float32)

        def p1(ji, carry):
            sacc, qacc, jv = carry
            tyrow = tybuf[ji]
            for u in range(16):
                tyv = tyrow.at[_vfull(u)].get(mode="promise_in_bounds")
                wv = plsc.load_gather(w, [tokv, jv])
                pv = plsc.load_gather(p, [tokv, jv])
                v = wv + pv + tyv
                plsc.store_scatter(o, [tokv, jv], v)
                sacc = sacc + v
                qacc = qacc + v * v
                jv = jv + onesv
            return sacc, qacc, jv

        sacc, qacc, _ = lax.fori_loop(0, H // 16, p1,
                                      (zf, zf, jnp.zeros((16,), jnp.int32)))
        mean = sacc * inv_h
        var = qacc * inv_h - mean * mean
        rinv = _rsqrt16(var + epsv)

        def p2(ji, jv):
            grow = gbuf[ji]
            brow = bbuf[ji]
            for u in range(16):
                gv = grow.at[_vfull(u)].get(mode="promise_in_bounds")
                bv = brow.at[_vfull(u)].get(mode="promise_in_bounds")
                v = plsc.load_gather(o, [tokv, jv])
                plsc.store_scatter(o, [tokv, jv],
                                   (v - mean) * rinv * gv + bv)
                jv = jv + onesv
            return jv

        lax.fori_loop(0, H // 16, p2, jnp.zeros((16,), jnp.int32))

    issue(0, 0)
    issue(1, 1)

    def pair_body(c2, _):
        for s in (0, 1):
            k = c2 * 2 + s
            wait_gather(k, s)
            off = pl.multiple_of(base + k * CHUNK, 8)

            @pl.when(k >= 1)
            def _wait_prev_out():
                pltpu.make_async_copy(
                    obuf, out_hbm.at[pl.ds(off, CHUNK)], osem).wait()

            compute_chunk(wrows[s], prows[s], obuf)
            pltpu.async_copy(obuf, out_hbm.at[pl.ds(off, CHUNK)], osem)

            @pl.when(c2 < NPAIR - 1)
            def _issue_next():
                issue(k + 2, s)
        return 0

    lax.fori_loop(0, NPAIR, pair_body, 0)

    off = pl.multiple_of(base + (NCHUNK - 1) * CHUNK, 8)
    pltpu.make_async_copy(
        obuf, out_hbm.at[pl.ds(off, CHUNK)], osem).wait()


@functools.partial(
    pl.kernel,
    out_type=jax.ShapeDtypeStruct((B * S, H), jnp.float32),
    mesh=plsc.VectorSubcoreMesh(core_axis_name="c", subcore_axis_name="s"),
    compiler_params=pltpu.CompilerParams(needs_layout_passes=False,
                                         use_tc_tiling_on_sc=False),
    scratch_types=[
        pltpu.VMEM((S // 16, 16), jnp.int32),
        pltpu.VMEM((NCHUNK, CHUNK), jnp.int32),
        pltpu.VMEM((NCHUNK, CHUNK), jnp.int32),
        pltpu.VMEM((CHUNK, H), jnp.float32),
        pltpu.VMEM((CHUNK, H), jnp.float32),
        pltpu.VMEM((CHUNK, H), jnp.float32),
        pltpu.VMEM((CHUNK, H), jnp.float32),
        pltpu.VMEM((CHUNK, H), jnp.float32),
        pltpu.VMEM((H // 16, 16), jnp.float32),
        pltpu.VMEM((H // 16, 16), jnp.float32),
        pltpu.VMEM((H // 16, 16), jnp.float32),
        pltpu.SemaphoreType.DMA,
        pltpu.SemaphoreType.DMA,
        pltpu.SemaphoreType.DMA,
        pltpu.SemaphoreType.DMA,
        pltpu.SemaphoreType.DMA,
    ],
)
def _sc_embed(ids_hbm, word_hbm, pos_hbm, ty_hbm, g_hbm, b_hbm, out_hbm,
              idsbuf, widx, pidx,
              wrows0, wrows1, prows0, prows1, obuf,
              tybuf, gbuf, bbuf,
              wsem0, wsem1, psem0, psem1, osem):
    _sc_body(ids_hbm, word_hbm, pos_hbm, ty_hbm, g_hbm, b_hbm, out_hbm,
             idsbuf, widx, pidx,
             wrows0, wrows1, prows0, prows1, obuf,
             tybuf, gbuf, bbuf,
             wsem0, wsem1, psem0, psem1, osem)


def kernel(input_ids, token_type_ids, word_emb, pos_emb, type_emb,
           ln_gamma, ln_beta):
    del token_type_ids
    ids2d = input_ids.reshape(B * S // 16, 16)
    out = _sc_embed(ids2d, word_emb, pos_emb,
                    type_emb.reshape(H // 16, 16),
                    ln_gamma.reshape(H // 16, 16),
                    ln_beta.reshape(H // 16, 16))
    return out.reshape(B, S, H)


# speedup vs baseline: 4.9704x; 4.9704x over previous
import functools

import jax
import jax.numpy as jnp
from jax import lax
from jax.experimental import pallas as pl
from jax.experimental.pallas import tpu as pltpu
from jax.experimental.pallas import tpu_sc as plsc

PAD = 1
EPS = 1e-5

B, S, H = 4, 4096, 1024
NW = 32
TOK_W = (B * S) // NW
SEG_PER_ROW = S // TOK_W
CHUNK = 16
NCHUNK = TOK_W // CHUNK
NPAIR = NCHUNK // 2
HC = H // 16


def _vfull(val, dtype=jnp.int32):
    return jnp.full((16,), val, dtype)


def _pad_mask(chunk):
    return jnp.minimum(jnp.abs(chunk - _vfull(PAD)), _vfull(1))


def _rsqrt16(x):
    i = plsc.bitcast(x, jnp.int32)
    i = _vfull(0x5F3759DF) - lax.shift_right_logical(i, _vfull(1))
    y = plsc.bitcast(i, jnp.float32)
    c15 = _vfull(1.5, jnp.float32)
    c05 = _vfull(0.5, jnp.float32)
    for _ in range(3):
        y = y * (c15 - c05 * x * y * y)
    return y


def _sc_body(ids_hbm, word_hbm, pos_hbm, g_hbm, b_hbm, out_hbm,
             idsbuf, widx, pidx,
             wrows0, wrows1, prows0, prows1, obuf0, obuf1,
             gbuf, bbuf,
             wsem0, wsem1, psem0, psem1, osem0, osem1):
    wid = lax.axis_index("c") * 16 + lax.axis_index("s")
    row = wid // SEG_PER_ROW
    seg = wid % SEG_PER_ROW

    wrows = (wrows0, wrows1)
    prows = (prows0, prows1)
    obuf = (obuf0, obuf1)
    wsem = (wsem0, wsem1)
    psem = (psem0, psem1)
    osem = (osem0, osem1)

    pltpu.sync_copy(g_hbm, gbuf)
    pltpu.sync_copy(b_hbm, bbuf)

    pltpu.sync_copy(ids_hbm.at[pl.ds(row * (S // 16), S // 16)], idsbuf)

    def pref_body(j, acc):
        return acc + _pad_mask(idsbuf[j])

    accv = lax.fori_loop(0, seg * (TOK_W // 16), pref_body,
                         jnp.zeros((16,), jnp.int32))
    prefv = jnp.broadcast_to(jnp.sum(accv), (16,))

    segbase = seg * (TOK_W // 16)
    for c in range(NCHUNK):
        chunk = idsbuf[segbase + c]
        m = _pad_mask(chunk)
        incl = plsc.cumsum(m)
        widx[c] = chunk
        pidx[c] = (prefv + incl) * m + _vfull(PAD)
        prefv = prefv + jnp.broadcast_to(jnp.sum(m), (16,))

    base = wid * TOK_W
    inv_h = jnp.float32(1.0 / H)
    epsv = _vfull(EPS, jnp.float32)

    def issue(k, s):
        pltpu.async_copy(word_hbm.at[widx.at[k]], wrows[s], wsem[s])
        pltpu.async_copy(pos_hbm.at[pidx.at[k]], prows[s], psem[s])

    def wait_gather(k, s):
        pltpu.make_async_copy(word_hbm.at[widx.at[k]], wrows[s], wsem[s]).wait()
        pltpu.make_async_copy(pos_hbm.at[pidx.at[k]], prows[s], psem[s]).wait()

    def compute_chunk(w, p, o):
        zf = jnp.zeros((16,), jnp.float32)
        meanv = []
        rinvv = []
        for t in range(CHUNK):
            def p1(ji, carry):
                sacc, qacc = carry
                for u in range(4):
                    sl = pl.ds((ji * 4 + u) * 16, 16)
                    v = w[t, sl] + p[t, sl]
                    o[t, sl] = v
                    sacc = sacc + v
                    qacc = qacc + v * v
                return sacc, qacc

            sacc, qacc = lax.fori_loop(0, HC // 4, p1, (zf, zf))
            mean = jnp.sum(sacc) * inv_h
            var = jnp.sum(qacc) * inv_h - mean * mean
            meanv.append(jnp.broadcast_to(mean, (16,)))
            rinvv.append(_rsqrt16(jnp.broadcast_to(var, (16,)) + epsv))

        for g0 in (0, 8):
            def p2(ji, _):
                gj = gbuf[ji]
                bj = bbuf[ji]
                sl = pl.ds(ji * 16, 16)
                for t in range(g0, g0 + 8):
                    v = o[t, sl]
                    o[t, sl] = (v - meanv[t]) * rinvv[t] * gj + bj
                return 0

            lax.fori_loop(0, HC, p2, 0)

    issue(0, 0)
    issue(1, 1)

    def pair_body(c2, _):
        for s in (0, 1):
            k = c2 * 2 + s
            wait_gather(k, s)
            off = pl.multiple_of(base + k * CHUNK, 8)

            @pl.when(c2 >= 1)
            def _wait_prev_out():
                pltpu.make_async_copy(
                    obuf[s], out_hbm.at[pl.ds(off, CHUNK)], osem[s]).wait()

            compute_chunk(wrows[s], prows[s], obuf[s])
            pltpu.async_copy(obuf[s], out_hbm.at[pl.ds(off, CHUNK)], osem[s])

            @pl.when(c2 < NPAIR - 1)
            def _issue_next():
                issue(k + 2, s)
        return 0

    lax.fori_loop(0, NPAIR, pair_body, 0)

    for s in (0, 1):
        k = NCHUNK - 2 + s
        off = pl.multiple_of(base + k * CHUNK, 8)
        pltpu.make_async_copy(
            obuf[s], out_hbm.at[pl.ds(off, CHUNK)], osem[s]).wait()


@functools.partial(
    pl.kernel,
    out_type=jax.ShapeDtypeStruct((B * S, H), jnp.float32),
    mesh=plsc.VectorSubcoreMesh(core_axis_name="c", subcore_axis_name="s"),
    compiler_params=pltpu.CompilerParams(needs_layout_passes=False,
                                         use_tc_tiling_on_sc=False),
    scratch_types=[
        pltpu.VMEM((S // 16, 16), jnp.int32),
        pltpu.VMEM((NCHUNK, CHUNK), jnp.int32),
        pltpu.VMEM((NCHUNK, CHUNK), jnp.int32),
        pltpu.VMEM((CHUNK, H), jnp.float32),
        pltpu.VMEM((CHUNK, H), jnp.float32),
        pltpu.VMEM((CHUNK, H), jnp.float32),
        pltpu.VMEM((CHUNK, H), jnp.float32),
        pltpu.VMEM((CHUNK, H), jnp.float32),
        pltpu.VMEM((CHUNK, H), jnp.float32),
        pltpu.VMEM((HC, 16), jnp.float32),
        pltpu.VMEM((HC, 16), jnp.float32),
        pltpu.SemaphoreType.DMA,
        pltpu.SemaphoreType.DMA,
        pltpu.SemaphoreType.DMA,
        pltpu.SemaphoreType.DMA,
        pltpu.SemaphoreType.DMA,
        pltpu.SemaphoreType.DMA,
    ],
)
def _sc_embed(ids_hbm, word_hbm, pos_hbm, g_hbm, b_hbm, out_hbm,
              idsbuf, widx, pidx,
              wrows0, wrows1, prows0, prows1, obuf0, obuf1,
              gbuf, bbuf,
              wsem0, wsem1, psem0, psem1, osem0, osem1):
    _sc_body(ids_hbm, word_hbm, pos_hbm, g_hbm, b_hbm, out_hbm,
             idsbuf, widx, pidx,
             wrows0, wrows1, prows0, prows1, obuf0, obuf1,
             gbuf, bbuf,
             wsem0, wsem1, psem0, psem1, osem0, osem1)


def kernel(input_ids, token_type_ids, word_emb, pos_emb, type_emb,
           ln_gamma, ln_beta):
    del token_type_ids
    ids2d = input_ids.reshape(B * S // 16, 16)
    pos_fold = pos_emb + type_emb.reshape(1, H)
    out = _sc_embed(ids2d, word_emb, pos_fold,
                    ln_gamma.reshape(HC, 16),
                    ln_beta.reshape(HC, 16))
    return out.reshape(B, S, H)
